# baseline (speedup 1.0000x reference)
import jax
import jax.numpy as jnp
from jax.experimental import pallas as pl
from jax.experimental.pallas import tpu as pltpu


def _fused_body(adj_ref, x_ref, Wt_ref, b_ref, a_ref, mwt_ref, mb_ref,
                mu_ref, sd_ref):
    G = adj_ref[0]
    x = x_ref[0]
    L = Wt_ref.shape[0]
    for li in range(L):
        mx = jnp.dot(x, Wt_ref[li], preferred_element_type=jnp.float32)
        mx = mx + b_ref[li][None, :]
        xo = jnp.dot(G, mx, preferred_element_type=jnp.float32)
        a = a_ref[0, li]
        xo = jnp.where(xo >= 0, xo, a * xo)
        x = xo + x
    y = jnp.dot(x, mwt_ref[...], preferred_element_type=jnp.float32)
    y = y + mb_ref[0][None, :]
    mix = y.shape[1]
    mu = jnp.sum(y, axis=1) / mix
    d = y - mu[:, None]
    var = jnp.sum(d * d, axis=1) / (mix - 1)
    mu_ref[0, 0] = mu
    sd_ref[0, 0] = jnp.sqrt(var)


def kernel(adj, vect_feat, input_mask, input_idx, adj_oh, gml_W, gml_b,
           gml_prelu, mix_W, mix_b):
    B, GS, N, _ = adj.shape
    F = vect_feat.shape[-1]
    L = gml_W.shape[0]
    MIX, OUT = mix_W.shape[0], mix_W.shape[1]

    adj2 = adj.reshape(B, N, N)
    Wt = jnp.swapaxes(gml_W.reshape(L, F, F), 1, 2)
    b = gml_b.reshape(L, F)
    a = gml_prelu.reshape(1, L)
    mwt = mix_W.reshape(MIX, F).T
    mb = mix_b.reshape(1, MIX)

    mu, sd = pl.pallas_call(
        _fused_body,
        grid=(B,),
        in_specs=[
            pl.BlockSpec((1, N, N), lambda i: (i, 0, 0)),
            pl.BlockSpec((1, N, F), lambda i: (i, 0, 0)),
            pl.BlockSpec((L, F, F), lambda i: (0, 0, 0)),
            pl.BlockSpec((L, F), lambda i: (0, 0)),
            pl.BlockSpec((1, L), lambda i: (0, 0)),
            pl.BlockSpec((F, MIX), lambda i: (0, 0)),
            pl.BlockSpec((1, MIX), lambda i: (0, 0)),
        ],
        out_specs=[
            pl.BlockSpec((1, 1, N), lambda i: (i, 0, 0)),
            pl.BlockSpec((1, 1, N), lambda i: (i, 0, 0)),
        ],
        out_shape=[
            jax.ShapeDtypeStruct((B, 1, N), jnp.float32),
            jax.ShapeDtypeStruct((B, 1, N), jnp.float32),
        ],
        compiler_params=pltpu.CompilerParams(
            dimension_semantics=("parallel",),
        ),
    )(adj2, vect_feat, Wt, b, a, mwt, mb)

    return mu.reshape(B, N, OUT), sd.reshape(B, N, OUT)
